# baseline (speedup 1.0000x reference)
import functools

import jax
import jax.numpy as jnp
from jax import lax
from jax.experimental import pallas as pl
from jax.experimental.pallas import tpu as pltpu
from jax.experimental.pallas import tpu_sc as plsc

_NC = 2
_NS = 16
_L = 16


def _make_kernel(B, K, b_per_w):
    mesh = plsc.VectorSubcoreMesh(core_axis_name="c", subcore_axis_name="s")

    @functools.partial(
        pl.kernel,
        mesh=mesh,
        compiler_params=pltpu.CompilerParams(
            needs_layout_passes=False, use_tc_tiling_on_sc=False
        ),
        out_type=jax.ShapeDtypeStruct((B,), jnp.float32),
        scratch_types=[
            pltpu.VMEM((b_per_w,), jnp.int32),
            pltpu.VMEM((b_per_w,), jnp.int32),
            pltpu.VMEM((b_per_w * K,), jnp.float32),
            pltpu.VMEM((b_per_w * K,), jnp.float32),
            pltpu.VMEM((b_per_w,), jnp.float32),
            pltpu.SemaphoreType.DMA,
            pltpu.SemaphoreType.DMA,
        ],
    )
    def mf(vids_hbm, mids_hbm, vtab_t, mtab_t, out_hbm,
           vidx, midx, ubuf, vbuf, outv, sem_u, sem_v):
        wid = lax.axis_index("s") * _NC + lax.axis_index("c")
        base = wid * b_per_w
        pltpu.sync_copy(vids_hbm.at[pl.ds(base, b_per_w)], vidx)
        pltpu.sync_copy(mids_hbm.at[pl.ds(base, b_per_w)], midx)

        for k in range(K):
            pltpu.async_copy(
                vtab_t.at[k].at[vidx], ubuf.at[pl.ds(k * b_per_w, b_per_w)],
                sem_u,
            )
            pltpu.async_copy(
                mtab_t.at[k].at[midx], vbuf.at[pl.ds(k * b_per_w, b_per_w)],
                sem_v,
            )
        pltpu.make_async_copy(
            vtab_t.at[0, pl.ds(0, b_per_w * K)], ubuf, sem_u
        ).wait()
        pltpu.make_async_copy(
            mtab_t.at[0, pl.ds(0, b_per_w * K)], vbuf, sem_v
        ).wait()

        def dot_body(g, _):
            b0 = g * _L
            acc = ubuf[pl.ds(b0, _L)] * vbuf[pl.ds(b0, _L)]
            for k in range(1, K):
                o = k * b_per_w + b0
                acc = acc + ubuf[pl.ds(o, _L)] * vbuf[pl.ds(o, _L)]
            outv[pl.ds(b0, _L)] = acc
            return 0

        lax.fori_loop(0, b_per_w // _L, dot_body, 0)
        pltpu.sync_copy(outv, out_hbm.at[pl.ds(base, b_per_w)])

    return mf


def kernel(viewer_ids, movie_ids, viewer_table, movie_table):
    B = viewer_ids.shape[0]
    K = viewer_table.shape[1]
    b_per_w = B // (_NC * _NS)
    mf = _make_kernel(B, K, b_per_w)
    vt = jnp.swapaxes(viewer_table, 0, 1)
    mt = jnp.swapaxes(movie_table, 0, 1)
    return mf(viewer_ids, movie_ids, vt, mt)


# speedup vs baseline: 4.6885x; 4.6885x over previous
import functools

import jax
import jax.numpy as jnp
from jax import lax
from jax.experimental import pallas as pl
from jax.experimental.pallas import tpu as pltpu
from jax.experimental.pallas import tpu_sc as plsc

_NC = 2
_NS = 16
_L = 16
_W = 128


def _make_kernel(B, K, b_per_w):
    mesh = plsc.VectorSubcoreMesh(core_axis_name="c", subcore_axis_name="s")
    rpp = _W // K

    @functools.partial(
        pl.kernel,
        mesh=mesh,
        compiler_params=pltpu.CompilerParams(needs_layout_passes=False),
        out_type=jax.ShapeDtypeStruct((B,), jnp.float32),
        scratch_types=[
            pltpu.VMEM((b_per_w,), jnp.int32),
            pltpu.VMEM((b_per_w,), jnp.int32),
            pltpu.VMEM((b_per_w,), jnp.int32),
            pltpu.VMEM((b_per_w,), jnp.int32),
            pltpu.VMEM((b_per_w,), jnp.int32),
            pltpu.VMEM((b_per_w,), jnp.int32),
            pltpu.VMEM((b_per_w // 2, _W), jnp.float32),
            pltpu.VMEM((b_per_w // 2, _W), jnp.float32),
            pltpu.VMEM((b_per_w,), jnp.float32),
            pltpu.SemaphoreType.DMA,
            pltpu.SemaphoreType.DMA,
        ],
    )
    def mf(vids_hbm, mids_hbm, vtab_p, mtab_p, out_hbm,
           vidx, midx, vrow, mrow, vsub, msub, ubuf, vbuf, outv,
           sem_u, sem_v):
        wid = lax.axis_index("s") * _NC + lax.axis_index("c")
        base = wid * b_per_w
        pltpu.sync_copy(vids_hbm.at[pl.ds(base, b_per_w)], vidx)
        pltpu.sync_copy(mids_hbm.at[pl.ds(base, b_per_w)], midx)

        def split_body(i, _):
            sl = pl.ds(i * _L, _L)
            v = vidx[sl]
            m = midx[sl]
            vrow[sl] = lax.shift_right_logical(v, 2)
            mrow[sl] = lax.shift_right_logical(m, 2)
            vsub[sl] = lax.mul(lax.bitwise_and(v, rpp - 1), K)
            msub[sl] = lax.mul(lax.bitwise_and(m, rpp - 1), K)
            return 0

        lax.fori_loop(0, b_per_w // _L, split_body, 0)

        lanes = lax.iota(jnp.int32, _L)
        half = b_per_w // 2

        for c in range(2):
            cu = pltpu.async_copy(
                vtab_p.at[vrow.at[pl.ds(c * half, half)]], ubuf, sem_u
            )
            cv = pltpu.async_copy(
                mtab_p.at[mrow.at[pl.ds(c * half, half)]], vbuf, sem_v
            )
            cu.wait()
            cv.wait()

            def dot_body(g, _):
                b0 = g * _L
                sl = pl.ds(c * half + b0, _L)
                ridx = b0 + lanes
                us = vsub[sl]
                ms = msub[sl]
                gu = plsc.load_gather(ubuf, [ridx, us])
                gv = plsc.load_gather(vbuf, [ridx, ms])
                acc = gu * gv
                for k in range(1, K):
                    gu = plsc.load_gather(ubuf, [ridx, us + k])
                    gv = plsc.load_gather(vbuf, [ridx, ms + k])
                    acc = acc + gu * gv
                outv[sl] = acc
                return 0

            lax.fori_loop(0, half // _L, dot_body, 0)
        pltpu.sync_copy(outv, out_hbm.at[pl.ds(base, b_per_w)])

    return mf


def kernel(viewer_ids, movie_ids, viewer_table, movie_table):
    B = viewer_ids.shape[0]
    K = viewer_table.shape[1]
    b_per_w = B // (_NC * _NS)
    mf = _make_kernel(B, K, b_per_w)
    vp = viewer_table.reshape(-1, _W)
    mp = movie_table.reshape(-1, _W)
    return mf(viewer_ids, movie_ids, vp, mp)
